# baseline (speedup 1.0000x reference)
import functools

import jax
import jax.numpy as jnp
from jax import lax
from jax.experimental import pallas as pl
from jax.experimental.pallas import tpu as pltpu
from jax.experimental.pallas import tpu_sc as plsc

EMB = 64
BATCH = 16384

NC = 2
NS = 16
NW = NC * NS
B_PER_W = BATCH // NW


def _sc_gather_all(u2, i2, ug_t, ig_t, um_t, im_t):
    mesh = plsc.VectorSubcoreMesh(core_axis_name="c", subcore_axis_name="s")
    row_t = jax.ShapeDtypeStruct((BATCH, 2 * EMB), jnp.float32)

    @functools.partial(
        pl.kernel,
        mesh=mesh,
        out_type=(row_t, row_t, row_t, row_t),
        scratch_types=[
            pltpu.VMEM((B_PER_W,), jnp.int32),
            pltpu.VMEM((B_PER_W,), jnp.int32),
            pltpu.VMEM((B_PER_W, 2 * EMB), jnp.float32),
            pltpu.SemaphoreType.DMA,
        ],
    )
    def k(u_hbm, i_hbm, ugt, igt, umt, imt,
          og_hbm, oi_hbm, om_hbm, oim_hbm,
          uidx_v, iidx_v, rows_v, sem):
        wid = lax.axis_index("s") * NC + lax.axis_index("c")
        base = wid * B_PER_W
        sl = pl.ds(base, B_PER_W)
        pltpu.sync_copy(u_hbm.at[sl], uidx_v)
        pltpu.sync_copy(i_hbm.at[sl], iidx_v)

        pltpu.async_copy(ugt.at[uidx_v], rows_v, sem).wait()
        pltpu.sync_copy(rows_v, og_hbm.at[sl])
        pltpu.async_copy(igt.at[iidx_v], rows_v, sem).wait()
        pltpu.sync_copy(rows_v, oi_hbm.at[sl])
        pltpu.async_copy(umt.at[uidx_v], rows_v, sem).wait()
        pltpu.sync_copy(rows_v, om_hbm.at[sl])
        pltpu.async_copy(imt.at[iidx_v], rows_v, sem).wait()
        pltpu.sync_copy(rows_v, oim_hbm.at[sl])

    return k(u2, i2, ug_t, ig_t, um_t, im_t)


BR = 1024


def _tc_dense_body(pu_ref, pi_ref, ug_ref, ig_ref, um_ref, im_ref,
                   w1a_ref, w1b_ref, b1_ref, w2_ref, b2_ref, w3_ref, b3_ref,
                   wmg_ref, wmh_ref, bm_ref, out_ref):
    f32 = jnp.float32
    pu = pu_ref[...] > 0
    pi = pi_ref[...] > 0

    def half(ref, par):
        return jnp.where(par, ref[:, EMB:], ref[:, :EMB])

    ug = half(ug_ref, pu)
    ig = half(ig_ref, pi)
    um = half(um_ref, pu)
    im = half(im_ref, pi)

    h1 = jnp.dot(um, w1a_ref[...], preferred_element_type=f32)
    h1 += jnp.dot(im, w1b_ref[...], preferred_element_type=f32)
    h1 = jnp.maximum(h1 + b1_ref[...], 0.0)
    h2 = jnp.maximum(
        jnp.dot(h1, w2_ref[...], preferred_element_type=f32) + b2_ref[...], 0.0)
    h3 = jnp.maximum(
        jnp.dot(h2, w3_ref[...], preferred_element_type=f32) + b3_ref[...], 0.0)
    g = ug * ig
    r = jnp.dot(g, wmg_ref[...], preferred_element_type=f32)
    r += jnp.dot(h3, wmh_ref[...], preferred_element_type=f32)
    out_ref[...] = r + bm_ref[...]


def _tc_dense(pu, pi, ug, ig, um, im, W1, b1, W2, b2, W3, b3, Wm, bm):
    w1a = W1[:, :EMB].T
    w1b = W1[:, EMB:].T
    w2 = W2.T
    w3 = W3.T
    wmg = Wm[:, :EMB].T
    wmh = Wm[:, EMB:].T
    b1r = b1.reshape(1, -1)
    b2r = b2.reshape(1, -1)
    b3r = b3.reshape(1, -1)
    bmr = bm.reshape(1, 1)

    par_spec = pl.BlockSpec((BR, 1), lambda i: (i, 0))
    row_spec = pl.BlockSpec((BR, 2 * EMB), lambda i: (i, 0))
    full = lambda a: pl.BlockSpec(a.shape, lambda i: (0,) * a.ndim)

    out = pl.pallas_call(
        _tc_dense_body,
        grid=(BATCH // BR,),
        in_specs=[par_spec, par_spec, row_spec, row_spec, row_spec, row_spec,
                  full(w1a), full(w1b), full(b1r), full(w2), full(b2r),
                  full(w3), full(b3r), full(wmg), full(wmh), full(bmr)],
        out_specs=pl.BlockSpec((BR, 1), lambda i: (i, 0)),
        out_shape=jax.ShapeDtypeStruct((BATCH, 1), jnp.float32),
    )(pu, pi, ug, ig, um, im, w1a, w1b, b1r, w2, b2r, w3, b3r, wmg, wmh, bmr)
    return jnp.squeeze(out, axis=-1)


def kernel(users, items, user_GMF, item_GMF, user_MLP, item_MLP,
           W1, b1, W2, b2, W3, b3, Wm, bm):
    u2 = lax.shift_right_logical(users, 1)
    i2 = lax.shift_right_logical(items, 1)
    pu = (users & 1).reshape(BATCH, 1)
    pi = (items & 1).reshape(BATCH, 1)
    ug_t = user_GMF.reshape(-1, 2 * EMB)
    ig_t = item_GMF.reshape(-1, 2 * EMB)
    um_t = user_MLP.reshape(-1, 2 * EMB)
    im_t = item_MLP.reshape(-1, 2 * EMB)
    ug, ig, um, im = _sc_gather_all(u2, i2, ug_t, ig_t, um_t, im_t)
    return _tc_dense(pu, pi, ug, ig, um, im, W1, b1, W2, b2, W3, b3, Wm, bm)


# speedup vs baseline: 1.0090x; 1.0090x over previous
import functools

import jax
import jax.numpy as jnp
from jax import lax
from jax.experimental import pallas as pl
from jax.experimental.pallas import tpu as pltpu
from jax.experimental.pallas import tpu_sc as plsc

EMB = 64
BATCH = 16384

NC = 2
NS = 16
NW = NC * NS
B_PER_W = BATCH // NW


def _sc_gather_all(users, items, ug_t, ig_t, um_t, im_t):
    mesh = plsc.VectorSubcoreMesh(core_axis_name="c", subcore_axis_name="s")
    row_t = jax.ShapeDtypeStruct((BATCH, EMB), jnp.float32)

    @functools.partial(
        pl.kernel,
        mesh=mesh,
        out_type=(row_t, row_t, row_t, row_t),
        compiler_params=pltpu.CompilerParams(use_tc_tiling_on_sc=False),
        scratch_types=[
            pltpu.VMEM((B_PER_W,), jnp.int32),
            pltpu.VMEM((B_PER_W,), jnp.int32),
            pltpu.VMEM((B_PER_W, EMB), jnp.float32),
            pltpu.VMEM((B_PER_W, EMB), jnp.float32),
            pltpu.SemaphoreType.DMA,
            pltpu.SemaphoreType.DMA,
            pltpu.SemaphoreType.DMA,
            pltpu.SemaphoreType.DMA,
        ],
    )
    def k(u_hbm, i_hbm, ugt, igt, umt, imt,
          og_hbm, oi_hbm, om_hbm, oim_hbm,
          uidx_v, iidx_v, rows_a, rows_b, gsa, gsb, wsa, wsb):
        wid = lax.axis_index("s") * NC + lax.axis_index("c")
        base = wid * B_PER_W
        sl = pl.ds(base, B_PER_W)
        pltpu.sync_copy(u_hbm.at[sl], uidx_v)
        pltpu.sync_copy(i_hbm.at[sl], iidx_v)

        g1 = pltpu.async_copy(ugt.at[uidx_v], rows_a, gsa)
        g2 = pltpu.async_copy(igt.at[iidx_v], rows_b, gsb)
        g1.wait()
        w1 = pltpu.async_copy(rows_a, og_hbm.at[sl], wsa)
        g2.wait()
        w2 = pltpu.async_copy(rows_b, oi_hbm.at[sl], wsb)
        w1.wait()
        g3 = pltpu.async_copy(umt.at[uidx_v], rows_a, gsa)
        w2.wait()
        g4 = pltpu.async_copy(imt.at[iidx_v], rows_b, gsb)
        g3.wait()
        w3 = pltpu.async_copy(rows_a, om_hbm.at[sl], wsa)
        g4.wait()
        w4 = pltpu.async_copy(rows_b, oim_hbm.at[sl], wsb)
        w3.wait()
        w4.wait()

    return k(users, items, ug_t, ig_t, um_t, im_t)


BR = 1024


def _tc_dense_body(ug_ref, ig_ref, um_ref, im_ref,
                   w1a_ref, w1b_ref, b1_ref, w2_ref, b2_ref, w3_ref, b3_ref,
                   wmg_ref, wmh_ref, bm_ref, out_ref):
    f32 = jnp.float32
    ug = ug_ref[...]
    ig = ig_ref[...]
    um = um_ref[...]
    im = im_ref[...]

    h1 = jnp.dot(um, w1a_ref[...], preferred_element_type=f32)
    h1 += jnp.dot(im, w1b_ref[...], preferred_element_type=f32)
    h1 = jnp.maximum(h1 + b1_ref[...], 0.0)
    h2 = jnp.maximum(
        jnp.dot(h1, w2_ref[...], preferred_element_type=f32) + b2_ref[...], 0.0)
    h3 = jnp.maximum(
        jnp.dot(h2, w3_ref[...], preferred_element_type=f32) + b3_ref[...], 0.0)
    g = ug * ig
    r = jnp.dot(g, wmg_ref[...], preferred_element_type=f32)
    r += jnp.dot(h3, wmh_ref[...], preferred_element_type=f32)
    out_ref[...] = r + bm_ref[...]


def _tc_dense(ug, ig, um, im, W1, b1, W2, b2, W3, b3, Wm, bm):
    w1a = W1[:, :EMB].T
    w1b = W1[:, EMB:].T
    w2 = W2.T
    w3 = W3.T
    wmg = Wm[:, :EMB].T
    wmh = Wm[:, EMB:].T
    b1r = b1.reshape(1, -1)
    b2r = b2.reshape(1, -1)
    b3r = b3.reshape(1, -1)
    bmr = bm.reshape(1, 1)

    row_spec = pl.BlockSpec((BR, EMB), lambda i: (i, 0))
    full = lambda a: pl.BlockSpec(a.shape, lambda i: (0,) * a.ndim)

    out = pl.pallas_call(
        _tc_dense_body,
        grid=(BATCH // BR,),
        in_specs=[row_spec, row_spec, row_spec, row_spec,
                  full(w1a), full(w1b), full(b1r), full(w2), full(b2r),
                  full(w3), full(b3r), full(wmg), full(wmh), full(bmr)],
        out_specs=pl.BlockSpec((BR, 1), lambda i: (i, 0)),
        out_shape=jax.ShapeDtypeStruct((BATCH, 1), jnp.float32),
    )(ug, ig, um, im, w1a, w1b, b1r, w2, b2r, w3, b3r, wmg, wmh, bmr)
    return jnp.squeeze(out, axis=-1)


def kernel(users, items, user_GMF, item_GMF, user_MLP, item_MLP,
           W1, b1, W2, b2, W3, b3, Wm, bm):
    ug, ig, um, im = _sc_gather_all(users, items, user_GMF, item_GMF,
                                    user_MLP, item_MLP)
    return _tc_dense(ug, ig, um, im, W1, b1, W2, b2, W3, b3, Wm, bm)


# speedup vs baseline: 1.2346x; 1.2236x over previous
import functools

import jax
import jax.numpy as jnp
from jax import lax
from jax.experimental import pallas as pl
from jax.experimental.pallas import tpu as pltpu
from jax.experimental.pallas import tpu_sc as plsc

EMB = 64
BATCH = 16384

NC = 2
NS = 16
NW = NC * NS
B_PER_W = BATCH // NW
CHUNK = 128


def _sc_gather2(users, items, u_tab, i_tab):
    mesh = plsc.VectorSubcoreMesh(core_axis_name="c", subcore_axis_name="s")
    row_t = jax.ShapeDtypeStruct((BATCH, 2 * EMB), jnp.float32)

    @functools.partial(
        pl.kernel,
        mesh=mesh,
        out_type=(row_t, row_t),
        scratch_types=[
            pltpu.VMEM((B_PER_W,), jnp.int32),
            pltpu.VMEM((B_PER_W,), jnp.int32),
            pltpu.VMEM((CHUNK, 2 * EMB), jnp.float32),
            pltpu.VMEM((CHUNK, 2 * EMB), jnp.float32),
            pltpu.SemaphoreType.DMA,
            pltpu.SemaphoreType.DMA,
        ],
    )
    def k(u_hbm, i_hbm, ut, it, ou_hbm, oi_hbm,
          uidx, iidx, bufa, bufb, gs0, gs1):
        wid = lax.axis_index("s") * NC + lax.axis_index("c")
        base = wid * B_PER_W
        sl = pl.ds(base, B_PER_W)
        pltpu.sync_copy(u_hbm.at[sl], uidx)
        pltpu.sync_copy(i_hbm.at[sl], iidx)

        nch = B_PER_W // CHUNK
        work = []
        for c in range(nch):
            work.append((ut, uidx, ou_hbm, c * CHUNK))
        for c in range(nch):
            work.append((it, iidx, oi_hbm, c * CHUNK))

        bufs = (bufa, bufb)
        gsems = (gs0, gs1)
        slots = [None, None]
        for n in (0, 1):
            tab, idx, _, off = work[n]
            slots[n] = pltpu.async_copy(
                tab.at[idx.at[pl.ds(off, CHUNK)]], bufs[n], gsems[n])
        for n in range(len(work)):
            b = n % 2
            _, _, out, off = work[n]
            slots[b].wait()
            pltpu.sync_copy(bufs[b], out.at[pl.ds(base + off, CHUNK)])
            if n + 2 < len(work):
                ntab, nidx, _, noff = work[n + 2]
                slots[b] = pltpu.async_copy(
                    ntab.at[nidx.at[pl.ds(noff, CHUNK)]], bufs[b], gsems[b])

    return k(users, items, u_tab, i_tab)


BR = 1024


def _tc_dense_body(u_ref, i_ref,
                   w1a_ref, w1b_ref, b1_ref, w2_ref, b2_ref, w3_ref, b3_ref,
                   wmg_ref, wmh_ref, bm_ref, out_ref):
    f32 = jnp.float32
    ug = u_ref[:, :EMB]
    um = u_ref[:, EMB:]
    ig = i_ref[:, :EMB]
    im = i_ref[:, EMB:]

    h1 = jnp.dot(um, w1a_ref[...], preferred_element_type=f32)
    h1 += jnp.dot(im, w1b_ref[...], preferred_element_type=f32)
    h1 = jnp.maximum(h1 + b1_ref[...], 0.0)
    h2 = jnp.maximum(
        jnp.dot(h1, w2_ref[...], preferred_element_type=f32) + b2_ref[...], 0.0)
    h3 = jnp.maximum(
        jnp.dot(h2, w3_ref[...], preferred_element_type=f32) + b3_ref[...], 0.0)
    g = ug * ig
    r = jnp.dot(g, wmg_ref[...], preferred_element_type=f32)
    r += jnp.dot(h3, wmh_ref[...], preferred_element_type=f32)
    out_ref[...] = r + bm_ref[...]


def _tc_dense(u_rows, i_rows, W1, b1, W2, b2, W3, b3, Wm, bm):
    w1a = W1[:, :EMB].T
    w1b = W1[:, EMB:].T
    w2 = W2.T
    w3 = W3.T
    wmg = Wm[:, :EMB].T
    wmh = Wm[:, EMB:].T
    b1r = b1.reshape(1, -1)
    b2r = b2.reshape(1, -1)
    b3r = b3.reshape(1, -1)
    bmr = bm.reshape(1, 1)

    row_spec = pl.BlockSpec((BR, 2 * EMB), lambda i: (i, 0))
    full = lambda a: pl.BlockSpec(a.shape, lambda i: (0,) * a.ndim)

    out = pl.pallas_call(
        _tc_dense_body,
        grid=(BATCH // BR,),
        in_specs=[row_spec, row_spec,
                  full(w1a), full(w1b), full(b1r), full(w2), full(b2r),
                  full(w3), full(b3r), full(wmg), full(wmh), full(bmr)],
        out_specs=pl.BlockSpec((BR, 1), lambda i: (i, 0)),
        out_shape=jax.ShapeDtypeStruct((BATCH, 1), jnp.float32),
    )(u_rows, i_rows, w1a, w1b, b1r, w2, b2r, w3, b3r, wmg, wmh, bmr)
    return jnp.squeeze(out, axis=-1)


def kernel(users, items, user_GMF, item_GMF, user_MLP, item_MLP,
           W1, b1, W2, b2, W3, b3, Wm, bm):
    u_tab = jnp.concatenate([user_GMF, user_MLP], axis=1)
    i_tab = jnp.concatenate([item_GMF, item_MLP], axis=1)
    u_rows, i_rows = _sc_gather2(users, items, u_tab, i_tab)
    return _tc_dense(u_rows, i_rows, W1, b1, W2, b2, W3, b3, Wm, bm)
